# baseline (speedup 1.0000x reference)
import functools
import numpy as np
import jax
import jax.numpy as jnp
from jax.experimental import pallas as pl
from jax.experimental.pallas import tpu as pltpu

IN_C, OUT_C, KSIZE, IMG = 3, 8, 5, 32
POOL_HW = 14
FC_OUT = 10
TB = 256
NK = 160
FLAT = IN_C * IMG * IMG
NF = 256


def _chan_weights_t(Wc):
    n = np.arange(NF)
    xpar = n // 128
    j = ((n % 128) // OUT_C) % 16
    o = n % OUT_C
    k = np.arange(NK)
    yloc = k // IMG
    xin = k % IMG
    kx = xin[:, None] - (2 * j + xpar)[None, :]
    valid = (kx >= 0) & (kx < KSIZE) & (j[None, :] < POOL_HW)
    ws = []
    for c in range(IN_C):
        src = ((o[None, :] * IN_C + c) * KSIZE + yloc[:, None]) * KSIZE \
            + np.clip(kx, 0, KSIZE - 1)
        wb = jnp.where(jnp.asarray(valid), Wc.reshape(-1)[jnp.asarray(src)], 0.0)
        ws.append(wb.astype(jnp.bfloat16))
    return jnp.stack(ws)


def _fc_weight_t(Wf):
    w4 = Wf.reshape(FC_OUT, OUT_C, POOL_HW, POOL_HW)
    w5 = jnp.transpose(w4, (2, 3, 1, 0)).reshape(POOL_HW, 112, FC_OUT)
    w5 = jnp.pad(w5, ((0, 0), (0, 16), (0, 6)))
    return w5.astype(jnp.bfloat16)


_DN = (((1,), (0,)), ((), ()))


def _net_kernel(x_ref, wt_ref, bcl_ref, wf_ref, bfl_ref, out_ref):
    xb = x_ref[0]
    wts = [wt_ref[0], wt_ref[1], wt_ref[2]]
    bcl = bcl_ref[...]
    accs = [jnp.zeros(out_ref.shape, jnp.float32) for _ in range(2)]
    for i in range(POOL_HW):
        rs = []
        for ypar in range(2):
            y0 = (2 * i + ypar) * IMG
            r = None
            for c in range(IN_C):
                xs = xb[:, c * IMG * IMG + y0:c * IMG * IMG + y0 + NK]
                d = jax.lax.dot_general(xs, wts[c], _DN,
                                        preferred_element_type=jnp.float32)
                r = d if r is None else r + d
            rs.append(r)
        ry = jnp.maximum(rs[0], rs[1])
        m = jnp.maximum(ry[:, :128], ry[:, 128:])
        a = jnp.maximum(m + bcl, 0.0).astype(jnp.bfloat16)
        accs[i % 2] = accs[i % 2] + jax.lax.dot_general(
            a, wf_ref[i], _DN, preferred_element_type=jnp.float32)
    out_ref[...] = accs[0] + accs[1] + bfl_ref[...]


@jax.jit
def _forward(x, Wc, bc, Wf, bf):
    B = x.shape[0]
    grid = pl.cdiv(B, TB)
    Bp = grid * TB
    if Bp != B:
        x = jnp.pad(x, ((0, Bp - B), (0, 0), (0, 0), (0, 0)))
    xr = x.reshape(Bp, FLAT).astype(jnp.bfloat16).reshape(grid, TB, FLAT)

    wt = _chan_weights_t(Wc)
    bcn = np.zeros((1, 128), np.float32)
    bcl = jnp.asarray(bcn) + jnp.tile(bc.astype(jnp.float32), 16).reshape(1, 128)
    wf_r = _fc_weight_t(Wf)
    bfl = jnp.pad(bf.astype(jnp.float32), (0, 6)).reshape(1, 16)

    flops = 2 * Bp * POOL_HW * (6 * NK * NF + 128 * 16)
    bytes_accessed = grid * TB * FLAT * 2 + 3 * NK * NF * 2 + Bp * 16 * 4

    out = pl.pallas_call(
        _net_kernel,
        out_shape=jax.ShapeDtypeStruct((Bp, 16), jnp.float32),
        grid=(grid,),
        in_specs=[
            pl.BlockSpec((1, TB, FLAT), lambda b: (b, 0, 0)),
            pl.BlockSpec((IN_C, NK, NF), lambda b: (0, 0, 0)),
            pl.BlockSpec((1, 128), lambda b: (0, 0)),
            pl.BlockSpec((POOL_HW, 128, 16), lambda b: (0, 0, 0)),
            pl.BlockSpec((1, 16), lambda b: (0, 0)),
        ],
        out_specs=pl.BlockSpec((TB, 16), lambda b: (b, 0)),
        compiler_params=pltpu.CompilerParams(
            dimension_semantics=("parallel",),
        ),
        cost_estimate=pl.CostEstimate(flops=int(flops), transcendentals=0,
                                      bytes_accessed=int(bytes_accessed)),
    )(xr, wt, bcl, wf_r, bfl)
    return out[:B, :FC_OUT]


def kernel(x, Wc, bc, Wf, bf):
    return _forward(x, Wc, bc, Wf, bf)
